# baseline (speedup 1.0000x reference)
import functools

import numpy as np
import jax
import jax.numpy as jnp
from jax import lax
from jax.experimental import pallas as pl
from jax.experimental.pallas import tpu as pltpu
from jax.experimental.pallas import tpu_sc as plsc

_H, _W = 300, 190
_N = _H * _W
_B = 64
_HP, _WP = _H + 2, _W + 2
_FLAT = _HP * _WP
_NPIX_I = _H * _WP
_L = 32
_LP = _L + 1
_ACC_ROW = 1104
_ACC_FLAT = 4 * _ACC_ROW
_BINS = np.linspace(-3.0, 3.0, 31).astype(np.float32)
_DELTAS = (1, _WP + 1, _WP, _WP - 1)


def _digitize_tree(xv):
    b = _BINS.tolist()
    step = jnp.float32(np.float32(0.2))
    zf = jnp.float32(0.0)

    c = xv >= b[15]
    bf = jnp.where(c, jnp.float32(16.0), zf)
    for sub, inc in ((8.0, 8.0), (12.0, 4.0), (14.0, 2.0)):
        t = (bf - jnp.float32(sub)) * step
        c = xv >= t
        bf = bf + jnp.where(c, jnp.float32(inc), zf)
    t = (bf - jnp.float32(15.0)) * step
    for kfix in (2, 6, 24, 28):
        t = jnp.where(bf == jnp.float32(kfix), jnp.float32(b[kfix]), t)
    c = xv >= t
    bf = bf + jnp.where(c, jnp.float32(1.0), zf)
    return bf.astype(jnp.int32)


_CHUNK = 75


def _stage1_digitize(x_ref, q_ref, s_ref):
    ssum = jnp.float32(0.0)
    ssq = jnp.float32(0.0)
    smax = jnp.float32(-jnp.inf)
    smin = jnp.float32(jnp.inf)
    for c in range(_H // _CHUNK):
        xv = x_ref[0, c * _CHUNK:(c + 1) * _CHUNK, :]
        q = _digitize_tree(xv)
        q_ref[0, c * _CHUNK:(c + 1) * _CHUNK, :] = jnp.concatenate(
            [q, jnp.full((_CHUNK, 2), _L, jnp.int32)], axis=1)
        ssum = ssum + jnp.sum(xv)
        ssq = ssq + jnp.sum(xv * xv)
        smax = jnp.maximum(smax, jnp.max(xv))
        smin = jnp.minimum(smin, jnp.min(xv))
    q_ref[0, _H:_HP, :] = jnp.full((2, _WP), _L, jnp.int32)
    n = jnp.float32(_N)
    m = ssum / n
    sd = jnp.sqrt(jnp.maximum(ssq / n - m * m, 0.0))
    mx = (smax - m) / sd
    mn = (m - smin) / sd
    z = jnp.float32(0.0)
    s_ref[...] = jnp.stack([m, sd, mx, mn, z, z, z, z]).reshape(1, 1, 8)


@functools.cache
def _make_stage2():
    mesh = plsc.VectorSubcoreMesh(
        core_axis_name="c", subcore_axis_name="s", num_cores=2, num_subcores=16)
    return pl.kernel(
        _stage2_scatter,
        out_type=[
            jax.ShapeDtypeStruct((_B, _ACC_FLAT), jnp.float32),
            jax.ShapeDtypeStruct((_B, 48), jnp.float32),
        ],

        mesh=mesh,
        scratch_types=[
            pltpu.VMEM((_FLAT,), jnp.int32),
            pltpu.VMEM((_ACC_FLAT,), jnp.float32),
            pltpu.VMEM((_ACC_FLAT,), jnp.float32),
            pltpu.VMEM((48,), jnp.float32),
        ],
        compiler_params=pltpu.CompilerParams(needs_layout_passes=False),
    )


def _stage2_scatter(qp_hbm, sym_hbm, hist_hbm, qbuf, acc, sym, hacc):
    wid = lax.axis_index("s") * 2 + lax.axis_index("c")
    ones = jnp.ones((16,), jnp.float32)
    zeros = jnp.zeros((16,), jnp.float32)
    lanes = jnp.arange(16, dtype=jnp.int32)
    for t in range(2):
        img = wid * 2 + t
        pltpu.sync_copy(qp_hbm.at[img], qbuf)

        @plsc.parallel_loop(0, _ACC_FLAT, step=16)
        def _zero(i):
            acc[pl.ds(i, 16)] = zeros

        @plsc.parallel_loop(0, _NPIX_I, step=16, unroll=8)
        def _scan(base):
            qi = qbuf[pl.ds(base, 16)]
            qi33 = qi * _LP
            for k, d in enumerate(_DELTAS):
                qj = qbuf[pl.ds(base + d, 16)]
                plsc.addupdate_scatter(acc, [qi33 + qj + k * _ACC_ROW], ones)

        for k in range(4):
            koff = k * _ACC_ROW

            @plsc.parallel_loop(0, _ACC_ROW, step=16, unroll=2)
            def _symm(c):
                cc = lanes + c
                ti = cc // _LP
                tj = cc - ti * _LP
                tidx = jnp.minimum(tj * _LP + ti, _LP * _LP - 1) + koff
                sym[pl.ds(c + koff, 16)] = (
                    acc[pl.ds(c + koff, 16)] + plsc.load_gather(acc, [tidx]))

        for c3 in range(3):
            v33 = (lanes + c3 * 16) * _LP
            h = jnp.zeros((16,), jnp.float32)
            for col in range(_LP):
                h = h + plsc.load_gather(acc, [v33 + col])
            hacc[pl.ds(c3 * 16, 16)] = h

        pltpu.sync_copy(sym, sym_hbm.at[img])
        pltpu.sync_copy(hacc, hist_hbm.at[img])


def _stage3_props(g_ref, h_ref, p_ref, hn_ref):
    idx = lax.broadcasted_iota(jnp.int32, (1, _ACC_ROW), 1)
    fi = (idx // _LP).astype(jnp.float32)
    fj = (idx % _LP).astype(jnp.float32)
    valid = (idx < _LP * _LP) & (idx // _LP < _L) & (idx % _LP < _L)

    g = jnp.where(valid, g_ref[...], 0.0)
    s = jnp.sum(g, axis=1, keepdims=True)
    g = g / jnp.where(s > 0, s, 1.0)
    sums = jnp.sum(g, axis=1, keepdims=True)
    P = g / jnp.where(sums == 0, 1.0, sums)

    d2 = (fi - fj) ** 2
    contrast = jnp.sum(P * d2, axis=1, keepdims=True)
    homog = jnp.sum(P / (1.0 + d2), axis=1, keepdims=True)
    energy = jnp.sqrt(jnp.sum(P * P, axis=1, keepdims=True))
    mu_i = jnp.sum(fi * P, axis=1, keepdims=True)
    mu_j = jnp.sum(fj * P, axis=1, keepdims=True)
    std_i = jnp.sqrt(jnp.sum(P * (fi - mu_i) ** 2, axis=1, keepdims=True))
    std_j = jnp.sqrt(jnp.sum(P * (fj - mu_j) ** 2, axis=1, keepdims=True))
    cov = jnp.sum(P * (fi - mu_i) * (fj - mu_j), axis=1, keepdims=True)
    den = std_i * std_j
    corr = jnp.where((std_i < 1e-15) | (std_j < 1e-15), 1.0,
                     cov / jnp.where(den == 0, 1.0, den))
    entropy = -jnp.sum(g * jnp.log2(g + 1e-8), axis=1, keepdims=True)
    z = jnp.zeros_like(contrast)
    p_ref[...] = jnp.concatenate(
        [contrast, homog, energy, corr, entropy, z, z, z], axis=1)

    hc = h_ref[...]
    hidx = lax.broadcasted_iota(jnp.int32, (1, 48), 1)
    hc = jnp.where(hidx < _L, hc, 0.0)
    allsame = jnp.max(hc, axis=1, keepdims=True) >= jnp.float32(_N)
    hn_ref[...] = jnp.where(allsame, 0.0, hc / jnp.float32(_N))[:, :_L]


def kernel(x):
    x3 = x.reshape(_B, _H, _W)
    q, stats = pl.pallas_call(
        _stage1_digitize,
        grid=(_B,),
        in_specs=[pl.BlockSpec((1, _H, _W), lambda i: (i, 0, 0))],
        out_specs=[
            pl.BlockSpec((1, _HP, _WP), lambda i: (i, 0, 0)),
            pl.BlockSpec((1, 1, 8), lambda i: (i, 0, 0)),
        ],
        out_shape=[
            jax.ShapeDtypeStruct((_B, _HP, _WP), jnp.int32),
            jax.ShapeDtypeStruct((_B, 1, 8), jnp.float32),
        ],
    )(x3)
    stats = stats.reshape(_B, 8)
    qp = q.reshape(_B, _FLAT)

    sym, hist = _make_stage2()(qp)

    props, hn = pl.pallas_call(
        _stage3_props,
        in_specs=[
            pl.BlockSpec((_B * 4, _ACC_ROW), lambda: (0, 0)),
            pl.BlockSpec((_B, 48), lambda: (0, 0)),
        ],
        out_specs=[
            pl.BlockSpec((_B * 4, 8), lambda: (0, 0)),
            pl.BlockSpec((_B, _L), lambda: (0, 0)),
        ],
        out_shape=[
            jax.ShapeDtypeStruct((_B * 4, 8), jnp.float32),
            jax.ShapeDtypeStruct((_B, _L), jnp.float32),
        ],
    )(sym.reshape(_B * 4, _ACC_ROW), hist)

    pr = props.reshape(_B, 4, 8)
    out = jnp.concatenate(
        [stats[:, :4], hn,
         pr[:, :, 0], pr[:, :, 1], pr[:, :, 2], pr[:, :, 3], pr[:, :, 4]],
        axis=1)
    return out


# speedup vs baseline: 1.1494x; 1.1494x over previous
import functools

import numpy as np
import jax
import jax.numpy as jnp
from jax import lax
from jax.experimental import pallas as pl
from jax.experimental.pallas import tpu as pltpu
from jax.experimental.pallas import tpu_sc as plsc

_B = 64
_H, _W = 190, 300
_N = _H * _W
_HP, _WP = _H + 2, _W + 4
_FLAT = _HP * _WP
_NPIX_I = _H * _WP
_L = 32
_LP = _L + 1
_ACC_ROW = 1104
_ACC_FLAT = 4 * _ACC_ROW
_BINS = np.linspace(-3.0, 3.0, 31).astype(np.float32)
_DELTAS = (_WP, _WP + 1, 1, _WP - 1)


def _digitize_tree(xv):
    b = _BINS.tolist()
    step = jnp.float32(np.float32(0.2))
    zf = jnp.float32(0.0)

    c = xv >= b[15]
    bf = jnp.where(c, jnp.float32(16.0), zf)
    for sub, inc in ((8.0, 8.0), (12.0, 4.0), (14.0, 2.0)):
        t = (bf - jnp.float32(sub)) * step
        c = xv >= t
        bf = bf + jnp.where(c, jnp.float32(inc), zf)
    t = (bf - jnp.float32(15.0)) * step
    for kfix in (2, 6, 24, 28):
        t = jnp.where(bf == jnp.float32(kfix), jnp.float32(b[kfix]), t)
    c = xv >= t
    bf = bf + jnp.where(c, jnp.float32(1.0), zf)
    return bf.astype(jnp.int32)


_CHUNK = 38


def _stage1_digitize(x_ref, q_ref, s_ref):
    ssum = jnp.float32(0.0)
    ssq = jnp.float32(0.0)
    smax = jnp.float32(-jnp.inf)
    smin = jnp.float32(jnp.inf)
    for c in range(_H // _CHUNK):
        xv = x_ref[0, c * _CHUNK:(c + 1) * _CHUNK, :]
        q = _digitize_tree(xv)
        q_ref[0, c * _CHUNK:(c + 1) * _CHUNK, :] = jnp.concatenate(
            [q, jnp.full((_CHUNK, _WP - _W), _L, jnp.int32)], axis=1)
        ssum = ssum + jnp.sum(xv)
        ssq = ssq + jnp.sum(xv * xv)
        smax = jnp.maximum(smax, jnp.max(xv))
        smin = jnp.minimum(smin, jnp.min(xv))
    q_ref[0, _H:_HP, :] = jnp.full((2, _WP), _L, jnp.int32)
    n = jnp.float32(_N)
    m = ssum / n
    sd = jnp.sqrt(jnp.maximum(ssq / n - m * m, 0.0))
    mx = (smax - m) / sd
    mn = (m - smin) / sd
    z = jnp.float32(0.0)
    s_ref[...] = jnp.stack([m, sd, mx, mn, z, z, z, z]).reshape(1, 1, 8)


@functools.cache
def _make_stage2():
    mesh = plsc.VectorSubcoreMesh(
        core_axis_name="c", subcore_axis_name="s", num_cores=2, num_subcores=16)
    return pl.kernel(
        _stage2_scatter,
        out_type=[
            jax.ShapeDtypeStruct((_B, _ACC_FLAT), jnp.float32),
            jax.ShapeDtypeStruct((_B, 48), jnp.float32),
        ],

        mesh=mesh,
        scratch_types=[
            pltpu.VMEM((_FLAT,), jnp.int32),
            pltpu.VMEM((_ACC_FLAT,), jnp.float32),
            pltpu.VMEM((_ACC_FLAT,), jnp.float32),
            pltpu.VMEM((48,), jnp.float32),
        ],
        compiler_params=pltpu.CompilerParams(needs_layout_passes=False),
    )


def _stage2_scatter(qp_hbm, sym_hbm, hist_hbm, qbuf, acc, sym, hacc):
    wid = lax.axis_index("s") * 2 + lax.axis_index("c")
    ones = jnp.ones((16,), jnp.float32)
    zeros = jnp.zeros((16,), jnp.float32)
    lanes = jnp.arange(16, dtype=jnp.int32)
    for t in range(2):
        img = wid * 2 + t
        pltpu.sync_copy(qp_hbm.at[img], qbuf)

        @plsc.parallel_loop(0, _ACC_FLAT, step=16)
        def _zero(i):
            acc[pl.ds(i, 16)] = zeros

        @plsc.parallel_loop(0, _NPIX_I, step=16, unroll=8)
        def _scan(base):
            qi = qbuf[pl.ds(base, 16)]
            qi33 = qi * _LP
            for k, d in enumerate(_DELTAS):
                qj = qbuf[pl.ds(base + d, 16)]
                plsc.addupdate_scatter(acc, [qi33 + qj + k * _ACC_ROW], ones)

        for k in range(4):
            koff = k * _ACC_ROW

            @plsc.parallel_loop(0, _ACC_ROW, step=16, unroll=2)
            def _symm(c):
                cc = lanes + c
                ti = cc // _LP
                tj = cc - ti * _LP
                tidx = jnp.minimum(tj * _LP + ti, _LP * _LP - 1) + koff
                sym[pl.ds(c + koff, 16)] = (
                    acc[pl.ds(c + koff, 16)] + plsc.load_gather(acc, [tidx]))

        for c3 in range(3):
            v33 = (lanes + c3 * 16) * _LP
            h = jnp.zeros((16,), jnp.float32)
            for col in range(_LP):
                h = h + plsc.load_gather(acc, [v33 + col])
            hacc[pl.ds(c3 * 16, 16)] = h

        pltpu.sync_copy(sym, sym_hbm.at[img])
        pltpu.sync_copy(hacc, hist_hbm.at[img])


def _stage3_props(g_ref, h_ref, p_ref, hn_ref):
    idx = lax.broadcasted_iota(jnp.int32, (1, _ACC_ROW), 1)
    fi = (idx // _LP).astype(jnp.float32)
    fj = (idx % _LP).astype(jnp.float32)
    valid = (idx < _LP * _LP) & (idx // _LP < _L) & (idx % _LP < _L)

    g = jnp.where(valid, g_ref[...], 0.0)
    s = jnp.sum(g, axis=1, keepdims=True)
    g = g / jnp.where(s > 0, s, 1.0)
    sums = jnp.sum(g, axis=1, keepdims=True)
    P = g / jnp.where(sums == 0, 1.0, sums)

    d2 = (fi - fj) ** 2
    contrast = jnp.sum(P * d2, axis=1, keepdims=True)
    homog = jnp.sum(P / (1.0 + d2), axis=1, keepdims=True)
    energy = jnp.sqrt(jnp.sum(P * P, axis=1, keepdims=True))
    mu_i = jnp.sum(fi * P, axis=1, keepdims=True)
    mu_j = jnp.sum(fj * P, axis=1, keepdims=True)
    std_i = jnp.sqrt(jnp.sum(P * (fi - mu_i) ** 2, axis=1, keepdims=True))
    std_j = jnp.sqrt(jnp.sum(P * (fj - mu_j) ** 2, axis=1, keepdims=True))
    cov = jnp.sum(P * (fi - mu_i) * (fj - mu_j), axis=1, keepdims=True)
    den = std_i * std_j
    corr = jnp.where((std_i < 1e-15) | (std_j < 1e-15), 1.0,
                     cov / jnp.where(den == 0, 1.0, den))
    entropy = -jnp.sum(g * jnp.log2(g + 1e-8), axis=1, keepdims=True)
    z = jnp.zeros_like(contrast)
    p_ref[...] = jnp.concatenate(
        [contrast, homog, energy, corr, entropy, z, z, z], axis=1)

    hc = h_ref[...]
    hidx = lax.broadcasted_iota(jnp.int32, (1, 48), 1)
    hc = jnp.where(hidx < _L, hc, 0.0)
    allsame = jnp.max(hc, axis=1, keepdims=True) >= jnp.float32(_N)
    hn_ref[...] = jnp.where(allsame, 0.0, hc / jnp.float32(_N))[:, :_L]


def kernel(x):
    x3 = jnp.swapaxes(x.reshape(_B, _W, _H), 1, 2)
    q, stats = pl.pallas_call(
        _stage1_digitize,
        grid=(_B,),
        in_specs=[pl.BlockSpec((1, _H, _W), lambda i: (i, 0, 0))],
        out_specs=[
            pl.BlockSpec((1, _HP, _WP), lambda i: (i, 0, 0)),
            pl.BlockSpec((1, 1, 8), lambda i: (i, 0, 0)),
        ],
        out_shape=[
            jax.ShapeDtypeStruct((_B, _HP, _WP), jnp.int32),
            jax.ShapeDtypeStruct((_B, 1, 8), jnp.float32),
        ],
    )(x3)
    stats = stats.reshape(_B, 8)
    qp = q.reshape(_B, _FLAT)

    sym, hist = _make_stage2()(qp)

    props, hn = pl.pallas_call(
        _stage3_props,
        in_specs=[
            pl.BlockSpec((_B * 4, _ACC_ROW), lambda: (0, 0)),
            pl.BlockSpec((_B, 48), lambda: (0, 0)),
        ],
        out_specs=[
            pl.BlockSpec((_B * 4, 8), lambda: (0, 0)),
            pl.BlockSpec((_B, _L), lambda: (0, 0)),
        ],
        out_shape=[
            jax.ShapeDtypeStruct((_B * 4, 8), jnp.float32),
            jax.ShapeDtypeStruct((_B, _L), jnp.float32),
        ],
    )(sym.reshape(_B * 4, _ACC_ROW), hist)

    pr = props.reshape(_B, 4, 8)
    out = jnp.concatenate(
        [stats[:, :4], hn,
         pr[:, :, 0], pr[:, :, 1], pr[:, :, 2], pr[:, :, 3], pr[:, :, 4]],
        axis=1)
    return out


# speedup vs baseline: 1.3816x; 1.2020x over previous
import functools

import numpy as np
import jax
import jax.numpy as jnp
from jax import lax
from jax.experimental import pallas as pl
from jax.experimental.pallas import tpu as pltpu
from jax.experimental.pallas import tpu_sc as plsc

_B = 64
_H, _W = 190, 300
_N = _H * _W
_HP, _WP = _H + 2, _W + 4
_FLAT = _HP * _WP
_NPIX_I = _H * _WP
_L = 32
_LP = _L + 1
_ACC_ROW = 1104
_ACC_FLAT = 4 * _ACC_ROW
_BINS = np.linspace(-3.0, 3.0, 31).astype(np.float32)
_DELTAS = (_WP, _WP + 1, 1, _WP - 1)


def _digitize_tree(xv):
    b = _BINS.tolist()
    step = jnp.float32(np.float32(0.2))
    zf = jnp.float32(0.0)

    c = xv >= b[15]
    bf = jnp.where(c, jnp.float32(16.0), zf)
    for sub, inc in ((8.0, 8.0), (12.0, 4.0), (14.0, 2.0)):
        t = (bf - jnp.float32(sub)) * step
        c = xv >= t
        bf = bf + jnp.where(c, jnp.float32(inc), zf)
    t = (bf - jnp.float32(15.0)) * step
    for kfix in (2, 6, 24, 28):
        t = jnp.where(bf == jnp.float32(kfix), jnp.float32(b[kfix]), t)
    c = xv >= t
    bf = bf + jnp.where(c, jnp.float32(1.0), zf)
    return bf.astype(jnp.int32)


_CHUNK = 38


def _stage1_digitize(x_ref, q_ref, s_ref):
    ssum = jnp.float32(0.0)
    ssq = jnp.float32(0.0)
    smax = jnp.float32(-jnp.inf)
    smin = jnp.float32(jnp.inf)
    for c in range(_H // _CHUNK):
        xv = x_ref[0, c * _CHUNK:(c + 1) * _CHUNK, :]
        q = _digitize_tree(xv)
        q_ref[0, c * _CHUNK:(c + 1) * _CHUNK, :] = jnp.concatenate(
            [q, jnp.full((_CHUNK, _WP - _W), _L, jnp.int32)], axis=1)
        ssum = ssum + jnp.sum(xv)
        ssq = ssq + jnp.sum(xv * xv)
        smax = jnp.maximum(smax, jnp.max(xv))
        smin = jnp.minimum(smin, jnp.min(xv))
    q_ref[0, _H:_HP, :] = jnp.full((2, _WP), _L, jnp.int32)
    n = jnp.float32(_N)
    m = ssum / n
    sd = jnp.sqrt(jnp.maximum(ssq / n - m * m, 0.0))
    mx = (smax - m) / sd
    mn = (m - smin) / sd
    z = jnp.float32(0.0)
    s_ref[...] = jnp.stack([m, sd, mx, mn, z, z, z, z]).reshape(1, 1, 8)


@functools.cache
def _make_stage2(nb):
    mesh = plsc.VectorSubcoreMesh(
        core_axis_name="c", subcore_axis_name="s", num_cores=2, num_subcores=16)
    return pl.kernel(
        functools.partial(_stage2_scatter, nb),
        out_type=[
            jax.ShapeDtypeStruct((nb, _ACC_FLAT), jnp.float32),
            jax.ShapeDtypeStruct((nb, 48), jnp.float32),
        ],
        mesh=mesh,
        scratch_types=[
            pltpu.VMEM((_FLAT,), jnp.int32),
            pltpu.VMEM((_ACC_FLAT,), jnp.float32),
            pltpu.VMEM((_ACC_FLAT,), jnp.float32),
            pltpu.VMEM((48,), jnp.float32),
        ],
        compiler_params=pltpu.CompilerParams(needs_layout_passes=False),
    )


def _stage2_scatter(nb, qp_hbm, sym_hbm, hist_hbm, qbuf, acc, sym, hacc):
    wid = lax.axis_index("s") * 2 + lax.axis_index("c")
    ones = jnp.ones((16,), jnp.float32)
    zeros = jnp.zeros((16,), jnp.float32)
    lanes = jnp.arange(16, dtype=jnp.int32)
    for t in range(nb // 32):
        img = wid * (nb // 32) + t
        pltpu.sync_copy(qp_hbm.at[img], qbuf)

        @plsc.parallel_loop(0, _ACC_FLAT, step=16)
        def _zero(i):
            acc[pl.ds(i, 16)] = zeros

        @plsc.parallel_loop(0, _NPIX_I, step=16, unroll=8)
        def _scan(base):
            qi = qbuf[pl.ds(base, 16)]
            qi33 = qi * _LP
            for k, d in enumerate(_DELTAS):
                qj = qbuf[pl.ds(base + d, 16)]
                plsc.addupdate_scatter(acc, [qi33 + qj + k * _ACC_ROW], ones)

        for k in range(4):
            koff = k * _ACC_ROW

            @plsc.parallel_loop(0, _ACC_ROW, step=16, unroll=2)
            def _symm(c):
                cc = lanes + c
                ti = cc // _LP
                tj = cc - ti * _LP
                tidx = jnp.minimum(tj * _LP + ti, _LP * _LP - 1) + koff
                sym[pl.ds(c + koff, 16)] = (
                    acc[pl.ds(c + koff, 16)] + plsc.load_gather(acc, [tidx]))

        for c3 in range(3):
            v33 = (lanes + c3 * 16) * _LP
            h = jnp.zeros((16,), jnp.float32)
            for col in range(_LP):
                h = h + plsc.load_gather(acc, [v33 + col])
            hacc[pl.ds(c3 * 16, 16)] = h

        pltpu.sync_copy(sym, sym_hbm.at[img])
        pltpu.sync_copy(hacc, hist_hbm.at[img])


def _stage3_props(g_ref, h_ref, p_ref, hn_ref):
    idx = lax.broadcasted_iota(jnp.int32, (1, _ACC_ROW), 1)
    fi = (idx // _LP).astype(jnp.float32)
    fj = (idx % _LP).astype(jnp.float32)
    valid = (idx < _LP * _LP) & (idx // _LP < _L) & (idx % _LP < _L)

    g = jnp.where(valid, g_ref[...], 0.0)
    s = jnp.sum(g, axis=1, keepdims=True)
    g = g / jnp.where(s > 0, s, 1.0)
    sums = jnp.sum(g, axis=1, keepdims=True)
    P = g / jnp.where(sums == 0, 1.0, sums)

    d2 = (fi - fj) ** 2
    contrast = jnp.sum(P * d2, axis=1, keepdims=True)
    homog = jnp.sum(P / (1.0 + d2), axis=1, keepdims=True)
    energy = jnp.sqrt(jnp.sum(P * P, axis=1, keepdims=True))
    mu_i = jnp.sum(fi * P, axis=1, keepdims=True)
    mu_j = jnp.sum(fj * P, axis=1, keepdims=True)
    std_i = jnp.sqrt(jnp.sum(P * (fi - mu_i) ** 2, axis=1, keepdims=True))
    std_j = jnp.sqrt(jnp.sum(P * (fj - mu_j) ** 2, axis=1, keepdims=True))
    cov = jnp.sum(P * (fi - mu_i) * (fj - mu_j), axis=1, keepdims=True)
    den = std_i * std_j
    corr = jnp.where((std_i < 1e-15) | (std_j < 1e-15), 1.0,
                     cov / jnp.where(den == 0, 1.0, den))
    entropy = -jnp.sum(g * jnp.log2(g + 1e-8), axis=1, keepdims=True)
    z = jnp.zeros_like(contrast)
    p_ref[...] = jnp.concatenate(
        [contrast, homog, energy, corr, entropy, z, z, z], axis=1)

    hc = h_ref[...]
    hidx = lax.broadcasted_iota(jnp.int32, (1, 48), 1)
    hc = jnp.where(hidx < _L, hc, 0.0)
    allsame = jnp.max(hc, axis=1, keepdims=True) >= jnp.float32(_N)
    hn_ref[...] = jnp.where(allsame, 0.0, hc / jnp.float32(_N))[:, :_L]


def _half_pipeline(x3, h, nb):
    q, stats = pl.pallas_call(
        _stage1_digitize,
        grid=(nb,),
        in_specs=[pl.BlockSpec((1, _H, _W), lambda i: (i + h * nb, 0, 0))],
        out_specs=[
            pl.BlockSpec((1, _HP, _WP), lambda i: (i, 0, 0)),
            pl.BlockSpec((1, 1, 8), lambda i: (i, 0, 0)),
        ],
        out_shape=[
            jax.ShapeDtypeStruct((nb, _HP, _WP), jnp.int32),
            jax.ShapeDtypeStruct((nb, 1, 8), jnp.float32),
        ],
    )(x3)
    stats = stats.reshape(nb, 8)
    qp = q.reshape(nb, _FLAT)

    sym, hist = _make_stage2(nb)(qp)

    props, hn = pl.pallas_call(
        _stage3_props,
        in_specs=[
            pl.BlockSpec((nb * 4, _ACC_ROW), lambda: (0, 0)),
            pl.BlockSpec((nb, 48), lambda: (0, 0)),
        ],
        out_specs=[
            pl.BlockSpec((nb * 4, 8), lambda: (0, 0)),
            pl.BlockSpec((nb, _L), lambda: (0, 0)),
        ],
        out_shape=[
            jax.ShapeDtypeStruct((nb * 4, 8), jnp.float32),
            jax.ShapeDtypeStruct((nb, _L), jnp.float32),
        ],
    )(sym.reshape(nb * 4, _ACC_ROW), hist)

    pr = props.reshape(nb, 4, 8)
    return jnp.concatenate(
        [stats[:, :4], hn,
         pr[:, :, 0], pr[:, :, 1], pr[:, :, 2], pr[:, :, 3], pr[:, :, 4]],
        axis=1)


def kernel(x):
    x3 = jnp.swapaxes(x.reshape(_B, _W, _H), 1, 2)
    nb = _B // 2
    out0 = _half_pipeline(x3, 0, nb)
    out1 = _half_pipeline(x3, 1, nb)
    return jnp.concatenate([out0, out1], axis=0)
